# baseline (speedup 1.0000x reference)
import functools

import jax
import jax.numpy as jnp
from jax import lax
from jax.experimental import pallas as pl
from jax.experimental.pallas import tpu as pltpu
from jax.experimental.pallas import tpu_sc as plsc

DIM = 128
B = 4096 * 50
NW = 32
BPW = B // NW
CHUNK = 128
NCH = BPW // CHUNK
NBUF = 4

_mesh = plsc.VectorSubcoreMesh(core_axis_name="c", subcore_axis_name="s")


@functools.partial(
    pl.kernel,
    out_type=jax.ShapeDtypeStruct((B, DIM), jnp.float32),
    mesh=_mesh,
    scratch_types=[
        pltpu.VMEM((NCH, CHUNK), jnp.int32),
        pltpu.VMEM((CHUNK, DIM), jnp.float32),
        pltpu.VMEM((CHUNK, DIM), jnp.float32),
        pltpu.VMEM((CHUNK, DIM), jnp.float32),
        pltpu.VMEM((CHUNK, DIM), jnp.float32),
        pltpu.SemaphoreType.DMA,
        pltpu.SemaphoreType.DMA,
        pltpu.SemaphoreType.DMA,
        pltpu.SemaphoreType.DMA,
        pltpu.SemaphoreType.DMA,
        pltpu.SemaphoreType.DMA,
        pltpu.SemaphoreType.DMA,
        pltpu.SemaphoreType.DMA,
    ],
)
def _onehot_sc(x_hbm, table_hbm, out_hbm, idx2, r0, r1, r2, r3,
               g0, g1, g2, g3, s0, s1, s2, s3):
    rows = (r0, r1, r2, r3)
    gsem = (g0, g1, g2, g3)
    ssem = (s0, s1, s2, s3)
    wid = lax.axis_index("s") * 2 + lax.axis_index("c")
    base = wid * BPW

    pltpu.sync_copy(x_hbm.at[wid], idx2)

    def _start_gather(h, b):
        pltpu.async_copy(table_hbm.at[idx2.at[h]], rows[b], gsem[b])

    def _wait_gather(h, b):
        pltpu.make_async_copy(table_hbm.at[idx2.at[h]], rows[b],
                              gsem[b]).wait()

    def _start_store(h, b):
        pltpu.async_copy(rows[b],
                         out_hbm.at[pl.ds(base + h * CHUNK, CHUNK)], ssem[b])

    def _wait_store(h, b):
        pltpu.make_async_copy(rows[b],
                              out_hbm.at[pl.ds(base + h * CHUNK, CHUNK)],
                              ssem[b]).wait()

    def body(t, carry):
        for p in range(NBUF):
            h = NBUF * t + p

            @pl.when(h < NCH)
            def _():
                @pl.when(h >= NBUF)
                def _():
                    _wait_store(h - NBUF, p)
                _start_gather(h, p)

            hs = h - 2
            bs = (p + 2) % NBUF

            @pl.when((hs >= 0) & (hs < NCH))
            def _():
                _wait_gather(hs, bs)
                _start_store(hs, bs)
        return carry

    n_virt = NCH + 2
    lax.fori_loop(0, (n_virt + NBUF - 1) // NBUF, body, 0)

    for q in range(NBUF):
        h = NCH - NBUF + q
        _wait_store(h, h % NBUF)


def kernel(x, I):
    out = _onehot_sc(x.reshape(NW, NCH, CHUNK), I)
    return out.reshape(x.shape + (DIM,))


# speedup vs baseline: 1.9483x; 1.9483x over previous
import functools

import jax
import jax.numpy as jnp
from jax import lax
from jax.experimental import pallas as pl
from jax.experimental.pallas import tpu as pltpu
from jax.experimental.pallas import tpu_sc as plsc

DIM = 128
B = 4096 * 50
NW = 32
BPW = B // NW
CHUNK = 128
NCH = BPW // CHUNK
NBUF = 6
LANES = 16
BUFW = CHUNK * DIM

_mesh = plsc.VectorSubcoreMesh(core_axis_name="c", subcore_axis_name="s")


@functools.partial(
    pl.kernel,
    out_type=jax.ShapeDtypeStruct((B * DIM,), jnp.float32),
    mesh=_mesh,
    scratch_types=(
        [pltpu.VMEM((BUFW,), jnp.float32) for _ in range(NBUF)]
        + [pltpu.VMEM((NBUF, CHUNK), jnp.int32),
           pltpu.VMEM((NBUF, CHUNK), jnp.int32)]
        + [pltpu.SemaphoreType.DMA for _ in range(2 * NBUF)]
    ),
    compiler_params=pltpu.CompilerParams(needs_layout_passes=False),
)
def _onehot_sc(x_hbm, table_hbm, out_hbm, *refs):
    rows = refs[:NBUF]
    nidx, ooff = refs[NBUF], refs[NBUF + 1]
    isem = refs[NBUF + 2:2 * NBUF + 2]
    ssem = refs[2 * NBUF + 2:3 * NBUF + 2]
    wid = lax.axis_index("s") * 2 + lax.axis_index("c")
    base = wid * BPW

    ones_v = jnp.full((LANES,), 1.0, jnp.float32)
    zeros_v = jnp.full((LANES,), 0.0, jnp.float32)
    lane = lax.iota(jnp.int32, LANES)

    def _start_idx(h, b):
        pltpu.async_copy(x_hbm.at[pl.ds(base + h * CHUNK, CHUNK)],
                         nidx.at[b], isem[b])

    def _wait_idx(h, b):
        pltpu.make_async_copy(x_hbm.at[pl.ds(base + h * CHUNK, CHUNK)],
                              nidx.at[b], isem[b]).wait()

    def _start_store(h, b):
        pltpu.async_copy(rows[b],
                         out_hbm.at[pl.ds((base + h * CHUNK) * DIM, BUFW)],
                         ssem[b])

    def _wait_store(h, b):
        pltpu.make_async_copy(rows[b],
                              out_hbm.at[pl.ds((base + h * CHUNK) * DIM,
                                               BUFW)],
                              ssem[b]).wait()

    _start_idx(0, 0)
    _start_idx(1, 1)

    def body(t, carry):
        for p in range(NBUF):
            h = NBUF * t + p

            @pl.when(h < NCH)
            def _():
                @pl.when(h + 2 < NCH)
                def _():
                    _start_idx(h + 2, (p + 2) % NBUF)

                _wait_idx(h, p)

                @pl.when(h < NBUF)
                def _():
                    def zbody(i, c):
                        for u in range(8):
                            rows[p][pl.ds((i * 8 + u) * LANES, LANES)] = (
                                zeros_v)
                        return c
                    lax.fori_loop(0, BUFW // LANES // 8, zbody, 0)

                @pl.when(h >= NBUF)
                def _():
                    _wait_store(h - NBUF, p)
                    for j in range(CHUNK // LANES):
                        sl = pl.ds(j * LANES, LANES)
                        plsc.store_scatter(rows[p], [ooff[p, sl]], zeros_v)

                for j in range(CHUNK // LANES):
                    sl = pl.ds(j * LANES, LANES)
                    off = (lane + (j * LANES)) * DIM + nidx[p, sl]
                    plsc.store_scatter(rows[p], [off], ones_v)
                    ooff[p, sl] = off

                _start_store(h, p)
        return carry

    lax.fori_loop(0, (NCH + NBUF - 1) // NBUF, body, 0)

    for q in range(NBUF):
        h = NCH - NBUF + q
        _wait_store(h, h % NBUF)


def kernel(x, I):
    out = _onehot_sc(x.reshape(-1), I)
    return out.reshape(x.shape + (DIM,))
